# baseline (speedup 1.0000x reference)
import jax
import jax.numpy as jnp
from jax import lax
from jax.experimental import pallas as pl
from jax.experimental.pallas import tpu as pltpu
from jax.experimental.pallas import tpu_sc as plsc

VOCAB = 1000
BATCH = 4096
HIST = 20

_info = plsc.get_sparse_core_info()
NC = _info.num_cores
NS = _info.num_subcores
L = _info.num_lanes
NW = NC * NS
CW = BATCH // NW
GRP = CW // L
VS = 128
NTASK = -(-VOCAB // VS)
SIZES = [min(VS, VOCAB - t * VS) for t in range(NTASK)]


def _mh_body(xT_hbm, outT_hbm, xv, acc0, acc1, sem_x, sem0, sem1):
    c = lax.axis_index("c")
    s = lax.axis_index("s")
    wid = s * NC + c
    base = wid * CW

    stage = pltpu.async_copy(xT_hbm.at[:, pl.ds(base, CW)], xv, sem_x)

    zeros = jnp.zeros((L,), jnp.float32)
    ones = jnp.ones((L,), jnp.float32)
    lane_iota = lax.iota(jnp.int32, L)

    def zero_buf(acc):
        def _z(i, carry):
            for u in range(4):
                for t in range(CW // L):
                    acc[i * 4 + u, pl.ds(t * L, L)] = zeros
            return carry

        lax.fori_loop(0, VS // 4, _z, None)

    accs = (acc0, acc1)
    sems = (sem0, sem1)

    def scatter_task(acc, t, val):
        lo = t * VS
        size = SIZES[t]

        def _g(g, carry):
            for h in range(2):
                colv = lane_iota + (g * 2 + h) * L
                vs = [xv[j, pl.ds((g * 2 + h) * L, L)] for j in range(HIST)]
                rls = [v - lo for v in vs]
                ms = [
                    plsc.bitcast(rl, jnp.uint32) < jnp.uint32(size)
                    for rl in rls
                ]
                for rl, m in zip(rls, ms):
                    if val is None:
                        plsc.addupdate_scatter(acc, [rl, colv], ones, mask=m)
                    else:
                        plsc.store_scatter(acc, [rl, colv], val, mask=m)
            return carry

        lax.fori_loop(0, GRP // 2, _g, None)

    copies = [None] * NTASK
    for t in range(NTASK):
        acc = accs[t % 2]
        if t < 2:
            zero_buf(acc)
            if t == 0:
                stage.wait()
        else:
            copies[t - 2].wait()
            scatter_task(acc, t - 2, zeros)
        scatter_task(acc, t, None)
        copies[t] = pltpu.async_copy(
            acc.at[pl.ds(0, SIZES[t]), :],
            outT_hbm.at[pl.ds(t * VS, SIZES[t]), pl.ds(base, CW)],
            sems[t % 2],
        )
    copies[NTASK - 2].wait()
    copies[NTASK - 1].wait()


def kernel(x):
    outT = pl.kernel(
        _mh_body,
        out_type=jax.ShapeDtypeStruct((VOCAB, BATCH), jnp.float32),
        mesh=plsc.VectorSubcoreMesh(core_axis_name="c", subcore_axis_name="s"),
        scratch_types=[
            pltpu.VMEM((HIST, CW), jnp.int32),
            pltpu.VMEM((VS, CW), jnp.float32),
            pltpu.VMEM((VS, CW), jnp.float32),
            pltpu.SemaphoreType.DMA,
            pltpu.SemaphoreType.DMA,
            pltpu.SemaphoreType.DMA,
        ],
        compiler_params=pltpu.CompilerParams(
            needs_layout_passes=False, use_tc_tiling_on_sc=True
        ),
    )(x.T)
    return outT.T


# speedup vs baseline: 1.0753x; 1.0753x over previous
import jax
import jax.numpy as jnp
from jax import lax
from jax.experimental import pallas as pl
from jax.experimental.pallas import tpu as pltpu
from jax.experimental.pallas import tpu_sc as plsc

VOCAB = 1000
BATCH = 4096
HIST = 20

_info = plsc.get_sparse_core_info()
NC = _info.num_cores
NS = _info.num_subcores
L = _info.num_lanes
NW = NC * NS
CW = BATCH // NW
GRP = CW // L
VS = 128
NTASK = -(-VOCAB // VS)
SIZES = [min(VS, VOCAB - t * VS) for t in range(NTASK)]


def _mh_body(xT_hbm, outT_hbm, xv, acc0, acc1, sem_x, sem0, sem1):
    c = lax.axis_index("c")
    s = lax.axis_index("s")
    wid = s * NC + c
    base = wid * CW

    stage = pltpu.async_copy(xT_hbm.at[:, pl.ds(base, CW)], xv, sem_x)

    zeros = jnp.zeros((L,), jnp.float32)
    ones = jnp.ones((L,), jnp.float32)
    lane_iota = lax.iota(jnp.int32, L)

    def zero_buf(acc):
        def _z(i, carry):
            for u in range(4):
                for t in range(CW // L):
                    acc[i * 4 + u, pl.ds(t * L, L)] = zeros
            return carry

        lax.fori_loop(0, VS // 4, _z, None)

    accs = (acc0, acc1)
    sems = (sem0, sem1)

    def scatter_task(acc, t, val):
        lo = t * VS
        size = SIZES[t]

        def _g(g, carry):
            colv = lane_iota + g * L
            vs = [xv[j, pl.ds(g * L, L)] for j in range(HIST)]
            rls = [v - lo for v in vs]
            ms = [
                plsc.bitcast(rl, jnp.uint32) < jnp.uint32(size) for rl in rls
            ]
            for rl, m in zip(rls, ms):
                if val is None:
                    plsc.addupdate_scatter(acc, [rl, colv], ones, mask=m)
                else:
                    plsc.store_scatter(acc, [rl, colv], val, mask=m)
            return carry

        lax.fori_loop(0, GRP, _g, None)

    copies = [None] * NTASK
    for t in range(NTASK):
        acc = accs[t % 2]
        if t < 2:
            zero_buf(acc)
            if t == 0:
                stage.wait()
        else:
            copies[t - 2].wait()
            scatter_task(acc, t - 2, zeros)
        scatter_task(acc, t, None)
        copies[t] = pltpu.async_copy(
            acc.at[pl.ds(0, SIZES[t]), :],
            outT_hbm.at[pl.ds(t * VS, SIZES[t]), pl.ds(base, CW)],
            sems[t % 2],
        )
    copies[NTASK - 2].wait()
    copies[NTASK - 1].wait()


def kernel(x):
    outT = pl.kernel(
        _mh_body,
        out_type=jax.ShapeDtypeStruct((VOCAB, BATCH), jnp.float32),
        mesh=plsc.VectorSubcoreMesh(core_axis_name="c", subcore_axis_name="s"),
        scratch_types=[
            pltpu.VMEM((HIST, CW), jnp.int32),
            pltpu.VMEM((VS, CW), jnp.float32),
            pltpu.VMEM((VS, CW), jnp.float32),
            pltpu.SemaphoreType.DMA,
            pltpu.SemaphoreType.DMA,
            pltpu.SemaphoreType.DMA,
        ],
        compiler_params=pltpu.CompilerParams(
            needs_layout_passes=False, use_tc_tiling_on_sc=True
        ),
    )(x.T)
    return outT.T
